# baseline (speedup 1.0000x reference)
import jax
import jax.numpy as jnp
from jax.experimental import pallas as pl
from jax.experimental.pallas import tpu as pltpu

_TIME_STEPS = 1000
_EMBED_DIM = 256


def _bcast_kernel(t_ref, emb_ref, out_ref):
    del t_ref
    out_ref[...] = jnp.broadcast_to(emb_ref[0], out_ref.shape)


def kernel(x, t, embeddings):
    b, _, h, w = x.shape
    hw = h * w
    wblk = hw
    n_w = 1
    for cand in (4, 2, 8, 16, 7, 14, 28):
        if hw % cand == 0 and (hw // cand) % 128 == 0:
            wblk = hw // cand
            n_w = cand
            break
    t_arr = jnp.asarray(t, jnp.int32).reshape(1)
    emb3 = embeddings.reshape(_TIME_STEPS, _EMBED_DIM, 1)
    grid_spec = pltpu.PrefetchScalarGridSpec(
        num_scalar_prefetch=1,
        grid=(b, n_w),
        in_specs=[
            pl.BlockSpec((1, _EMBED_DIM, 1), lambda i, j, t_ref: (t_ref[0], 0, 0)),
        ],
        out_specs=pl.BlockSpec((_EMBED_DIM, wblk), lambda i, j, t_ref: (i, j)),
    )
    out2d = pl.pallas_call(
        _bcast_kernel,
        grid_spec=grid_spec,
        out_shape=jax.ShapeDtypeStruct((b * _EMBED_DIM, hw), jnp.float32),
    )(t_arr, emb3)
    return out2d.reshape(b, _EMBED_DIM, h, w)
